# Initial kernel scaffold; baseline (speedup 1.0000x reference)
import jax
import jax.numpy as jnp
from jax.experimental import pallas as pl


def kernel(x_static, x_dynamic, edge_index, params):
    raise NotImplementedError("write your pallas kernel here")


# speedup vs baseline: 4.7649x; 4.7649x over previous
import functools

import jax
import jax.numpy as jnp
from jax import lax
from jax.experimental import pallas as pl
from jax.experimental.pallas import tpu as pltpu, tpu_sc as plsc

N = 10000
E = 320000
STATIC_DIM = 128
DYN_DIM = 16
T = 24
GNN_H = 128
RNN_H = 64
RNN_DIM = 2 * RNN_H
FUSED = GNN_H + RNN_DIM
EPS = 1e-12

NC = 2
NS = 16
NW = NC * NS
EPW = E // NW
CH = 80
NCHUNK = EPW // CH
NPAD = 10240
ROWS_PER_TILE = NPAD // NS
DEG_PAD = 10240
HALF = 64

_SC_MESH = dict(core_axis_name="c", subcore_axis_name="s",
                num_cores=NC, num_subcores=NS)


def _zero_vmem_2d(ref, nrows, ncols):
    z = jnp.zeros((16,), jnp.float32)

    def body(i, _):
        r = i // (ncols // 16)
        k = i % (ncols // 16)
        ref[r, pl.ds(k * 16, 16)] = z
        return 0

    lax.fori_loop(0, nrows * (ncols // 16), body, 0)


def _zero_vmem_1d(ref, n):
    z = jnp.zeros((16,), jnp.float32)

    def body(i, _):
        ref[pl.ds(i * 16, 16)] = z
        return 0

    lax.fori_loop(0, n // 16, body, 0)


def _segsum_body(x_hbm, src_hbm, dst_hbm, agg_hbm,
                 idx_src, idx_dst, rows0, rows1, zbuf, agg_sh, sem0, sem1):
    c = lax.axis_index("c")
    s = lax.axis_index("s")
    w = s * NC + c

    _zero_vmem_2d(zbuf, CH, HALF)
    for r in range(ROWS_PER_TILE // CH):
        pltpu.sync_copy(zbuf, agg_sh.at[pl.ds(s * ROWS_PER_TILE + r * CH, CH)])
    plsc.subcore_barrier()

    pltpu.sync_copy(src_hbm.at[w], idx_src)
    pltpu.sync_copy(dst_hbm.at[w], idx_dst)

    def pair(p, _):
        j0 = 2 * p
        j1 = j0 + 1
        c0 = pltpu.async_copy(x_hbm.at[idx_src.at[j0]], rows0, sem0)
        c1 = pltpu.async_copy(x_hbm.at[idx_src.at[j1]], rows1, sem1)
        c0.wait()
        pltpu.sync_copy(rows0, agg_sh.at[idx_dst.at[j0]], add=True)
        c1.wait()
        pltpu.sync_copy(rows1, agg_sh.at[idx_dst.at[j1]], add=True)
        return 0

    lax.fori_loop(0, NCHUNK // 2, pair, 0)
    ct = pltpu.async_copy(x_hbm.at[idx_src.at[NCHUNK - 1]], rows0, sem0)
    ct.wait()
    pltpu.sync_copy(rows0, agg_sh.at[idx_dst.at[NCHUNK - 1]], add=True)

    plsc.subcore_barrier()
    pltpu.sync_copy(agg_sh.at[pl.ds(s * ROWS_PER_TILE, ROWS_PER_TILE)],
                    agg_hbm.at[c, pl.ds(s * ROWS_PER_TILE, ROWS_PER_TILE)])


def _sc_segsum(x, src3d, dst3d):
    k = pl.kernel(
        _segsum_body,
        out_type=jax.ShapeDtypeStruct((NC, NPAD, HALF), jnp.float32),
        mesh=plsc.VectorSubcoreMesh(**_SC_MESH),
        compiler_params=pltpu.CompilerParams(use_tc_tiling_on_sc=False),
        scratch_types=[
            pltpu.VMEM((NCHUNK, CH), jnp.int32),
            pltpu.VMEM((NCHUNK, CH), jnp.int32),
            pltpu.VMEM((CH, HALF), jnp.float32),
            pltpu.VMEM((CH, HALF), jnp.float32),
            pltpu.VMEM((CH, HALF), jnp.float32),
            pltpu.VMEM_SHARED((NPAD, HALF), jnp.float32),
            pltpu.SemaphoreType.DMA,
            pltpu.SemaphoreType.DMA,
        ],
    )
    return k(x, src3d, dst3d)


def _deg_body(dst_hbm, deg_hbm, idx_dst, ones_v, zb, deg_sh):
    c = lax.axis_index("c")
    s = lax.axis_index("s")
    w = s * NC + c

    _zero_vmem_1d(zb, DEG_PAD // NS)
    pltpu.sync_copy(zb, deg_sh.at[pl.ds(s * (DEG_PAD // NS), DEG_PAD // NS)])

    def ones_init(i, _):
        ones_v[pl.ds(i * 16, 16)] = jnp.ones((16,), jnp.float32)
        return 0

    lax.fori_loop(0, CH // 16, ones_init, 0)
    plsc.subcore_barrier()

    pltpu.sync_copy(dst_hbm.at[w], idx_dst)

    def body(j, _):
        pltpu.sync_copy(ones_v, deg_sh.at[idx_dst.at[j]], add=True)
        return 0

    lax.fori_loop(0, NCHUNK, body, 0)
    plsc.subcore_barrier()
    pltpu.sync_copy(deg_sh.at[pl.ds(s * (DEG_PAD // NS), DEG_PAD // NS)],
                    deg_hbm.at[c, pl.ds(s * (DEG_PAD // NS), DEG_PAD // NS)])


def _sc_degree(dst2d):
    k = pl.kernel(
        _deg_body,
        out_type=jax.ShapeDtypeStruct((NC, DEG_PAD), jnp.float32),
        mesh=plsc.VectorSubcoreMesh(**_SC_MESH),
        scratch_types=[
            pltpu.VMEM((NCHUNK, CH), jnp.int32),
            pltpu.VMEM((CH,), jnp.float32),
            pltpu.VMEM((DEG_PAD // NS,), jnp.float32),
            pltpu.VMEM_SHARED((DEG_PAD,), jnp.float32),
        ],
    )
    return k(dst2d)


BN = 400


def _dot(a, b):
    return jnp.dot(a, b, preferred_element_type=jnp.float32)


def _sage1_tc_body(agglo_ref, agghi_ref, degp_ref, x_ref,
                   wl_ref, wr_ref, b_ref, h_ref):
    agg = jnp.concatenate([agglo_ref[0] + agglo_ref[1],
                           agghi_ref[0] + agghi_ref[1]], axis=1)
    deg = degp_ref[:, 0] + degp_ref[:, 1]
    inv = 1.0 / jnp.maximum(deg, 1.0)
    mean = agg * inv[:, None]
    out = _dot(mean, wl_ref[...]) + _dot(x_ref[...], wr_ref[...]) + b_ref[...]
    nrm = jnp.sqrt(jnp.sum(out * out, axis=1, keepdims=True))
    out = out / jnp.maximum(nrm, EPS)
    h_ref[...] = jnp.maximum(out, 0.0)


def _tc_sage1(agglo, agghi, degp, x, wl, wr, b):
    full = lambda a: pl.BlockSpec(a.shape, lambda i: (0,) * a.ndim)
    return pl.pallas_call(
        _sage1_tc_body,
        grid=(N // BN,),
        in_specs=[
            pl.BlockSpec((NC, BN, HALF), lambda i: (0, i, 0)),
            pl.BlockSpec((NC, BN, HALF), lambda i: (0, i, 0)),
            pl.BlockSpec((BN, NC), lambda i: (i, 0)),
            pl.BlockSpec((BN, STATIC_DIM), lambda i: (i, 0)),
            full(wl), full(wr), full(b),
        ],
        out_specs=pl.BlockSpec((BN, GNN_H), lambda i: (i, 0)),
        out_shape=jax.ShapeDtypeStruct((N, GNN_H), jnp.float32),
    )(agglo, agghi, degp, x, wl, wr, b)


def _lstm_step(g, h, c, whh_t):
    g = g + _dot(h, whh_t)
    i = jax.nn.sigmoid(g[:, 0:RNN_H])
    f = jax.nn.sigmoid(g[:, RNN_H:2 * RNN_H])
    gg = jnp.tanh(g[:, 2 * RNN_H:3 * RNN_H])
    o = jax.nn.sigmoid(g[:, 3 * RNN_H:4 * RNN_H])
    c = f * c + i * gg
    h = o * jnp.tanh(c)
    return h, c


def _main_tc_body(agglo_ref, agghi_ref, degp_ref, h_ref, xd_ref,
                  w2l_ref, w2r_ref, b2_ref,
                  w1cat_ref, b1cat_ref, whh1f_ref, whh1b_ref,
                  wih2f_ref, b2f_ref, whh2f_ref, wih2b_ref, b2b_ref,
                  wqg_ref, bqg_ref, wkr_ref, bkr_ref, wvr_ref, bvr_ref,
                  wprg_ref,
                  wqr_ref, bqr_ref, wkg_ref, bkg_ref, wvg_ref, bvg_ref,
                  wpgr_ref,
                  w1m_ref, b1m_ref, w2m_ref, b2m_ref, w3m_ref, b3m_ref,
                  logits_ref, p0_ref, p1_ref):
    h = h_ref[...]
    agg = jnp.concatenate([agglo_ref[0] + agglo_ref[1],
                           agghi_ref[0] + agghi_ref[1]], axis=1)
    deg = degp_ref[:, 0] + degp_ref[:, 1]
    inv = 1.0 / jnp.maximum(deg, 1.0)
    mean = agg * inv[:, None]
    g = _dot(mean, w2l_ref[...]) + _dot(h, w2r_ref[...]) + b2_ref[...]
    nrm = jnp.sqrt(jnp.sum(g * g, axis=1, keepdims=True))
    h_gnn = g / jnp.maximum(nrm, EPS)

    xblk = xd_ref[...]
    w1cat = w1cat_ref[...]
    b1cat = b1cat_ref[...]
    g1 = [
        _dot(xblk[:, t * DYN_DIM:(t + 1) * DYN_DIM], w1cat) + b1cat
        for t in range(T)
    ]
    zero = jnp.zeros((xblk.shape[0], RNN_H), jnp.float32)

    whh1b = whh1b_ref[...]
    hb, cb = zero, zero
    hb_seq = [None] * T
    for t in range(T - 1, -1, -1):
        hb, cb = _lstm_step(g1[t][:, 4 * RNN_H:], hb, cb, whh1b)
        hb_seq[t] = hb

    whh1f = whh1f_ref[...]
    wih2f = wih2f_ref[...]
    b2f = b2f_ref[...]
    whh2f = whh2f_ref[...]
    hf, cf = zero, zero
    h2, c2 = zero, zero
    for t in range(T):
        hf, cf = _lstm_step(g1[t][:, :4 * RNN_H], hf, cf, whh1f)
        x2t = jnp.concatenate([hf, hb_seq[t]], axis=1)
        g2 = _dot(x2t, wih2f) + b2f
        h2, c2 = _lstm_step(g2, h2, c2, whh2f)
    x2_last = jnp.concatenate([hf, hb_seq[T - 1]], axis=1)
    g2b = _dot(x2_last, wih2b_ref[...]) + b2b_ref[...]
    ib = jax.nn.sigmoid(g2b[:, 0:RNN_H])
    ggb = jnp.tanh(g2b[:, 2 * RNN_H:3 * RNN_H])
    ob = jax.nn.sigmoid(g2b[:, 3 * RNN_H:4 * RNN_H])
    c2b = ib * ggb
    h2b = ob * jnp.tanh(c2b)
    h_rnn = jnp.concatenate([h2, h2b], axis=1)

    q_gnn = _dot(h_gnn, wqg_ref[...]) + bqg_ref[...]
    k_rnn = _dot(h_rnn, wkr_ref[...]) + bkr_ref[...]
    v_rnn = _dot(h_rnn, wvr_ref[...]) + bvr_ref[...]
    s_g = jnp.sum(q_gnn * k_rnn, axis=1, keepdims=True)
    a_g = jax.nn.sigmoid(s_g)
    h_gnn_att = h_gnn + a_g * _dot(v_rnn, wprg_ref[...])

    q_rnn = _dot(h_rnn, wqr_ref[...]) + bqr_ref[...]
    k_gnn = _dot(h_gnn, wkg_ref[...]) + bkg_ref[...]
    v_gnn = _dot(h_gnn, wvg_ref[...]) + bvg_ref[...]
    s_r = jnp.sum(q_rnn * k_gnn, axis=1, keepdims=True)
    a_r = jax.nn.sigmoid(s_r)
    h_rnn_att = h_rnn + a_r * _dot(v_gnn, wpgr_ref[...])

    p0 = jax.nn.sigmoid(a_g - a_r)
    p0_ref[...] = p0
    p1_ref[...] = 1.0 - p0

    z = jnp.concatenate([h_gnn_att, h_rnn_att], axis=1)
    z = jnp.maximum(_dot(z, w1m_ref[...]) + b1m_ref[...], 0.0)
    z = jnp.maximum(_dot(z, w2m_ref[...]) + b2m_ref[...], 0.0)
    logits = _dot(z, w3m_ref[...]) + b3m_ref[...]
    logits_ref[...] = logits[:, 0:1]


def _tc_main(agglo2, agghi2, degp, h, xd, weights):
    full = lambda a: pl.BlockSpec(a.shape, lambda i: (0,) * a.ndim)
    in_specs = [
        pl.BlockSpec((NC, BN, HALF), lambda i: (0, i, 0)),
        pl.BlockSpec((NC, BN, HALF), lambda i: (0, i, 0)),
        pl.BlockSpec((BN, NC), lambda i: (i, 0)),
        pl.BlockSpec((BN, GNN_H), lambda i: (i, 0)),
        pl.BlockSpec((BN, T * DYN_DIM), lambda i: (i, 0)),
    ] + [full(w) for w in weights]
    return pl.pallas_call(
        _main_tc_body,
        grid=(N // BN,),
        in_specs=in_specs,
        out_specs=[
            pl.BlockSpec((BN, 1), lambda i: (i, 0)),
            pl.BlockSpec((BN, 1), lambda i: (i, 0)),
            pl.BlockSpec((BN, 1), lambda i: (i, 0)),
        ],
        out_shape=[
            jax.ShapeDtypeStruct((N, 1), jnp.float32),
            jax.ShapeDtypeStruct((N, 1), jnp.float32),
            jax.ShapeDtypeStruct((N, 1), jnp.float32),
        ],
    )(agglo2, agghi2, degp, h, xd, *weights)


def _prep_weights(params):
    p = params
    w1l = p["sage1_l"]["W"].T
    w1r = p["sage1_r"]["W"].T
    b1 = (p["sage1_l"]["b"] + p["sage1_r"]["b"])[None, :]
    w2l = p["sage2_l"]["W"].T
    w2r = p["sage2_r"]["W"].T
    b2 = (p["sage2_l"]["b"] + p["sage2_r"]["b"])[None, :]

    l1f, l1b = p["lstm"][0]["fwd"], p["lstm"][0]["bwd"]
    l2f, l2b = p["lstm"][1]["fwd"], p["lstm"][1]["bwd"]
    w1cat = jnp.concatenate([l1f["Wih"].T, l1b["Wih"].T], axis=1)
    b1cat = jnp.concatenate([l1f["bih"] + l1f["bhh"],
                             l1b["bih"] + l1b["bhh"]])[None, :]
    whh1f = l1f["Whh"].T
    whh1b = l1b["Whh"].T
    wih2f = l2f["Wih"].T
    b2f = (l2f["bih"] + l2f["bhh"])[None, :]
    whh2f = l2f["Whh"].T
    wih2b = l2b["Wih"].T
    b2b = (l2b["bih"] + l2b["bhh"])[None, :]

    sg = 1.0 / jnp.sqrt(jnp.float32(GNN_H))
    sr = 1.0 / jnp.sqrt(jnp.float32(RNN_DIM))
    wqg = p["q_gnn"]["W"].T * sg
    bqg = (p["q_gnn"]["b"] * sg)[None, :]
    wkr = p["k_rnn"]["W"].T
    bkr = p["k_rnn"]["b"][None, :]
    wvr = p["v_rnn"]["W"].T
    bvr = p["v_rnn"]["b"][None, :]
    wprg = p["proj_rnn_to_gnn"]["W"].T
    wqr = p["q_rnn"]["W"].T * sr
    bqr = (p["q_rnn"]["b"] * sr)[None, :]
    wkg = p["k_gnn"]["W"].T
    bkg = p["k_gnn"]["b"][None, :]
    wvg = p["v_gnn"]["W"].T
    bvg = p["v_gnn"]["b"][None, :]
    wpgr = p["proj_gnn_to_rnn"]["W"].T

    s1 = p["bn1"]["gamma"] / jnp.sqrt(p["bn1"]["var"] + 1e-5)
    w1m = p["mlp1"]["W"].T * s1[None, :]
    b1m = ((p["mlp1"]["b"] - p["bn1"]["mean"]) * s1 + p["bn1"]["beta"])[None, :]
    s2 = p["bn2"]["gamma"] / jnp.sqrt(p["bn2"]["var"] + 1e-5)
    w2m = p["mlp2"]["W"].T * s2[None, :]
    b2m = ((p["mlp2"]["b"] - p["bn2"]["mean"]) * s2 + p["bn2"]["beta"])[None, :]
    w3m = jnp.zeros((64, 128), jnp.float32).at[:, 0].set(p["mlp3"]["W"][0])
    b3m = jnp.zeros((1, 128), jnp.float32).at[0, 0].set(p["mlp3"]["b"][0])

    sage1 = (w1l, w1r, b1)
    main = (w2l, w2r, b2,
            w1cat, b1cat, whh1f, whh1b,
            wih2f, b2f, whh2f, wih2b, b2b,
            wqg, bqg, wkr, bkr, wvr, bvr, wprg,
            wqr, bqr, wkg, bkg, wvg, bvg, wpgr,
            w1m, b1m, w2m, b2m, w3m, b3m)
    return sage1, main


def kernel(x_static, x_dynamic, edge_index, params):
    src3d = edge_index[0].reshape(NW, NCHUNK, CH)
    dst3d = edge_index[1].reshape(NW, NCHUNK, CH)
    sage1_w, main_w = _prep_weights(params)

    degp = _sc_degree(dst3d)
    degp = degp[:, :N].T
    agglo1 = _sc_segsum(x_static[:, :HALF], src3d, dst3d)
    agghi1 = _sc_segsum(x_static[:, HALF:], src3d, dst3d)
    h = _tc_sage1(agglo1, agghi1, degp, x_static, *sage1_w)
    agglo2 = _sc_segsum(h[:, :HALF], src3d, dst3d)
    agghi2 = _sc_segsum(h[:, HALF:], src3d, dst3d)
    xd = x_dynamic.reshape(N, T * DYN_DIM)
    logits, p0, p1 = _tc_main(agglo2, agghi2, degp, h, xd, main_w)
    attn = jnp.concatenate([p0, p1], axis=1)
    return logits[:, 0], attn


# speedup vs baseline: 5.2954x; 1.1113x over previous
import functools

import jax
import jax.numpy as jnp
from jax import lax
from jax.experimental import pallas as pl
from jax.experimental.pallas import tpu as pltpu, tpu_sc as plsc

N = 10000
E = 320000
STATIC_DIM = 128
DYN_DIM = 16
T = 24
GNN_H = 128
RNN_H = 64
RNN_DIM = 2 * RNN_H
FUSED = GNN_H + RNN_DIM
EPS = 1e-12

NC = 2
NS = 16
NW = NC * NS
EPS_SUB = E // NS
CH = 80
NCHUNK = EPS_SUB // CH
NPAD = 10240
ROWS_PER_TILE = NPAD // NS
DEG_PAD = 10240
HALF = 64

_SC_MESH = dict(core_axis_name="c", subcore_axis_name="s",
                num_cores=NC, num_subcores=NS)


def _zero_vmem_2d(ref, nrows, ncols):
    z = jnp.zeros((16,), jnp.float32)

    def body(i, _):
        r = i // (ncols // 16)
        k = i % (ncols // 16)
        ref[r, pl.ds(k * 16, 16)] = z
        return 0

    lax.fori_loop(0, nrows * (ncols // 16), body, 0)


def _zero_vmem_1d(ref, n):
    z = jnp.zeros((16,), jnp.float32)

    def body(i, _):
        ref[pl.ds(i * 16, 16)] = z
        return 0

    lax.fori_loop(0, n // 16, body, 0)


def _segsum_body(xs_hbm, src_hbm, dst_hbm, agg_hbm,
                 idx_src, idx_dst, rows0, rows1, zbuf, agg_sh, sem0, sem1):
    c = lax.axis_index("c")
    s = lax.axis_index("s")

    _zero_vmem_2d(zbuf, CH, HALF)
    for r in range(ROWS_PER_TILE // CH):
        pltpu.sync_copy(zbuf, agg_sh.at[pl.ds(s * ROWS_PER_TILE + r * CH, CH)])
    plsc.subcore_barrier()

    pltpu.sync_copy(src_hbm.at[s], idx_src)
    pltpu.sync_copy(dst_hbm.at[s], idx_dst)

    xh = xs_hbm.at[c]
    pltpu.async_copy(xh.at[idx_src.at[0]], rows0, sem0)

    def pair(p, _):
        j0 = 2 * p
        j1 = j0 + 1
        j2 = j0 + 2
        pltpu.async_copy(xh.at[idx_src.at[j1]], rows1, sem1)
        pltpu.make_async_copy(xh.at[idx_src.at[j0]], rows0, sem0).wait()
        pltpu.sync_copy(rows0, agg_sh.at[idx_dst.at[j0]], add=True)

        @pl.when(j2 < NCHUNK)
        def _():
            pltpu.async_copy(xh.at[idx_src.at[j2]], rows0, sem0)

        pltpu.make_async_copy(xh.at[idx_src.at[j1]], rows1, sem1).wait()
        pltpu.sync_copy(rows1, agg_sh.at[idx_dst.at[j1]], add=True)
        return 0

    lax.fori_loop(0, NCHUNK // 2, pair, 0)

    plsc.subcore_barrier()
    pltpu.sync_copy(agg_sh.at[pl.ds(s * ROWS_PER_TILE, ROWS_PER_TILE)],
                    agg_hbm.at[c, pl.ds(s * ROWS_PER_TILE, ROWS_PER_TILE)])


def _sc_segsum(xs, src3d, dst3d):
    k = pl.kernel(
        _segsum_body,
        out_type=jax.ShapeDtypeStruct((NC, NPAD, HALF), jnp.float32),
        mesh=plsc.VectorSubcoreMesh(**_SC_MESH),
        compiler_params=pltpu.CompilerParams(use_tc_tiling_on_sc=False),
        scratch_types=[
            pltpu.VMEM((NCHUNK, CH), jnp.int32),
            pltpu.VMEM((NCHUNK, CH), jnp.int32),
            pltpu.VMEM((CH, HALF), jnp.float32),
            pltpu.VMEM((CH, HALF), jnp.float32),
            pltpu.VMEM((CH, HALF), jnp.float32),
            pltpu.VMEM_SHARED((NPAD, HALF), jnp.float32),
            pltpu.SemaphoreType.DMA,
            pltpu.SemaphoreType.DMA,
        ],
    )
    return k(xs, src3d, dst3d)


def _deg_body(dst_hbm, deg_hbm, idx_dst, ones_v, zb, deg_sh):
    c = lax.axis_index("c")
    s = lax.axis_index("s")

    _zero_vmem_1d(zb, DEG_PAD // NS)
    pltpu.sync_copy(zb, deg_sh.at[pl.ds(s * (DEG_PAD // NS), DEG_PAD // NS)])

    def ones_init(i, _):
        ones_v[pl.ds(i * 16, 16)] = jnp.ones((16,), jnp.float32)
        return 0

    lax.fori_loop(0, CH // 16, ones_init, 0)
    plsc.subcore_barrier()

    pltpu.sync_copy(dst_hbm.at[s, pl.ds(c * (NCHUNK // 2), NCHUNK // 2)],
                    idx_dst)

    def body(j, _):
        pltpu.sync_copy(ones_v, deg_sh.at[idx_dst.at[j]], add=True)
        return 0

    lax.fori_loop(0, NCHUNK // 2, body, 0)
    plsc.subcore_barrier()
    pltpu.sync_copy(deg_sh.at[pl.ds(s * (DEG_PAD // NS), DEG_PAD // NS)],
                    deg_hbm.at[c, pl.ds(s * (DEG_PAD // NS), DEG_PAD // NS)])


def _sc_degree(dst3d):
    k = pl.kernel(
        _deg_body,
        out_type=jax.ShapeDtypeStruct((NC, DEG_PAD), jnp.float32),
        mesh=plsc.VectorSubcoreMesh(**_SC_MESH),
        compiler_params=pltpu.CompilerParams(use_tc_tiling_on_sc=False),
        scratch_types=[
            pltpu.VMEM((NCHUNK // 2, CH), jnp.int32),
            pltpu.VMEM((CH,), jnp.float32),
            pltpu.VMEM((DEG_PAD // NS,), jnp.float32),
            pltpu.VMEM_SHARED((DEG_PAD,), jnp.float32),
        ],
    )
    return k(dst3d)


BN = 400


def _dot(a, b):
    return jnp.dot(a, b, preferred_element_type=jnp.float32)


def _sage1_tc_body(aggp_ref, degp_ref, x_ref,
                   wl_ref, wr_ref, b_ref, h_ref):
    agg = jnp.concatenate([aggp_ref[0], aggp_ref[1]], axis=1)
    deg = degp_ref[:, 0] + degp_ref[:, 1]
    inv = 1.0 / jnp.maximum(deg, 1.0)
    mean = agg * inv[:, None]
    out = _dot(mean, wl_ref[...]) + _dot(x_ref[...], wr_ref[...]) + b_ref[...]
    nrm = jnp.sqrt(jnp.sum(out * out, axis=1, keepdims=True))
    out = out / jnp.maximum(nrm, EPS)
    h_ref[...] = jnp.maximum(out, 0.0)


def _tc_sage1(aggp, degp, x, wl, wr, b):
    full = lambda a: pl.BlockSpec(a.shape, lambda i: (0,) * a.ndim)
    return pl.pallas_call(
        _sage1_tc_body,
        grid=(N // BN,),
        in_specs=[
            pl.BlockSpec((NC, BN, HALF), lambda i: (0, i, 0)),
            pl.BlockSpec((BN, NC), lambda i: (i, 0)),
            pl.BlockSpec((BN, STATIC_DIM), lambda i: (i, 0)),
            full(wl), full(wr), full(b),
        ],
        out_specs=pl.BlockSpec((BN, GNN_H), lambda i: (i, 0)),
        out_shape=jax.ShapeDtypeStruct((N, GNN_H), jnp.float32),
    )(aggp, degp, x, wl, wr, b)


def _lstm_step(g, h, c, whh_t):
    g = g + _dot(h, whh_t)
    i = jax.nn.sigmoid(g[:, 0:RNN_H])
    f = jax.nn.sigmoid(g[:, RNN_H:2 * RNN_H])
    gg = jnp.tanh(g[:, 2 * RNN_H:3 * RNN_H])
    o = jax.nn.sigmoid(g[:, 3 * RNN_H:4 * RNN_H])
    c = f * c + i * gg
    h = o * jnp.tanh(c)
    return h, c


def _main_tc_body(aggp_ref, degp_ref, h_ref, xd_ref,
                  w2l_ref, w2r_ref, b2_ref,
                  w1cat_ref, b1cat_ref, whh1f_ref, whh1b_ref,
                  wih2f_ref, b2f_ref, whh2f_ref, wih2b_ref, b2b_ref,
                  wqg_ref, bqg_ref, wkr_ref, bkr_ref, wvr_ref, bvr_ref,
                  wprg_ref,
                  wqr_ref, bqr_ref, wkg_ref, bkg_ref, wvg_ref, bvg_ref,
                  wpgr_ref,
                  w1m_ref, b1m_ref, w2m_ref, b2m_ref, w3m_ref, b3m_ref,
                  logits_ref, p0_ref, p1_ref):
    h = h_ref[...]
    agg = jnp.concatenate([aggp_ref[0], aggp_ref[1]], axis=1)
    deg = degp_ref[:, 0] + degp_ref[:, 1]
    inv = 1.0 / jnp.maximum(deg, 1.0)
    mean = agg * inv[:, None]
    g = _dot(mean, w2l_ref[...]) + _dot(h, w2r_ref[...]) + b2_ref[...]
    nrm = jnp.sqrt(jnp.sum(g * g, axis=1, keepdims=True))
    h_gnn = g / jnp.maximum(nrm, EPS)

    xblk = xd_ref[...]
    w1cat = w1cat_ref[...]
    b1cat = b1cat_ref[...]
    g1 = [
        _dot(xblk[:, t * DYN_DIM:(t + 1) * DYN_DIM], w1cat) + b1cat
        for t in range(T)
    ]
    zero = jnp.zeros((xblk.shape[0], RNN_H), jnp.float32)

    whh1b = whh1b_ref[...]
    hb, cb = zero, zero
    hb_seq = [None] * T
    for t in range(T - 1, -1, -1):
        hb, cb = _lstm_step(g1[t][:, 4 * RNN_H:], hb, cb, whh1b)
        hb_seq[t] = hb

    whh1f = whh1f_ref[...]
    wih2f = wih2f_ref[...]
    b2f = b2f_ref[...]
    whh2f = whh2f_ref[...]
    hf, cf = zero, zero
    h2, c2 = zero, zero
    for t in range(T):
        hf, cf = _lstm_step(g1[t][:, :4 * RNN_H], hf, cf, whh1f)
        x2t = jnp.concatenate([hf, hb_seq[t]], axis=1)
        g2 = _dot(x2t, wih2f) + b2f
        h2, c2 = _lstm_step(g2, h2, c2, whh2f)
    x2_last = jnp.concatenate([hf, hb_seq[T - 1]], axis=1)
    g2b = _dot(x2_last, wih2b_ref[...]) + b2b_ref[...]
    ib = jax.nn.sigmoid(g2b[:, 0:RNN_H])
    ggb = jnp.tanh(g2b[:, 2 * RNN_H:3 * RNN_H])
    ob = jax.nn.sigmoid(g2b[:, 3 * RNN_H:4 * RNN_H])
    c2b = ib * ggb
    h2b = ob * jnp.tanh(c2b)
    h_rnn = jnp.concatenate([h2, h2b], axis=1)

    q_gnn = _dot(h_gnn, wqg_ref[...]) + bqg_ref[...]
    k_rnn = _dot(h_rnn, wkr_ref[...]) + bkr_ref[...]
    v_rnn = _dot(h_rnn, wvr_ref[...]) + bvr_ref[...]
    s_g = jnp.sum(q_gnn * k_rnn, axis=1, keepdims=True)
    a_g = jax.nn.sigmoid(s_g)
    h_gnn_att = h_gnn + a_g * _dot(v_rnn, wprg_ref[...])

    q_rnn = _dot(h_rnn, wqr_ref[...]) + bqr_ref[...]
    k_gnn = _dot(h_gnn, wkg_ref[...]) + bkg_ref[...]
    v_gnn = _dot(h_gnn, wvg_ref[...]) + bvg_ref[...]
    s_r = jnp.sum(q_rnn * k_gnn, axis=1, keepdims=True)
    a_r = jax.nn.sigmoid(s_r)
    h_rnn_att = h_rnn + a_r * _dot(v_gnn, wpgr_ref[...])

    p0 = jax.nn.sigmoid(a_g - a_r)
    p0_ref[...] = p0
    p1_ref[...] = 1.0 - p0

    z = jnp.concatenate([h_gnn_att, h_rnn_att], axis=1)
    z = jnp.maximum(_dot(z, w1m_ref[...]) + b1m_ref[...], 0.0)
    z = jnp.maximum(_dot(z, w2m_ref[...]) + b2m_ref[...], 0.0)
    logits = _dot(z, w3m_ref[...]) + b3m_ref[...]
    logits_ref[...] = logits[:, 0:1]


def _tc_main(aggp2, degp, h, xd, weights):
    full = lambda a: pl.BlockSpec(a.shape, lambda i: (0,) * a.ndim)
    in_specs = [
        pl.BlockSpec((NC, BN, HALF), lambda i: (0, i, 0)),
        pl.BlockSpec((BN, NC), lambda i: (i, 0)),
        pl.BlockSpec((BN, GNN_H), lambda i: (i, 0)),
        pl.BlockSpec((BN, T * DYN_DIM), lambda i: (i, 0)),
    ] + [full(w) for w in weights]
    return pl.pallas_call(
        _main_tc_body,
        grid=(N // BN,),
        in_specs=in_specs,
        out_specs=[
            pl.BlockSpec((BN, 1), lambda i: (i, 0)),
            pl.BlockSpec((BN, 1), lambda i: (i, 0)),
            pl.BlockSpec((BN, 1), lambda i: (i, 0)),
        ],
        out_shape=[
            jax.ShapeDtypeStruct((N, 1), jnp.float32),
            jax.ShapeDtypeStruct((N, 1), jnp.float32),
            jax.ShapeDtypeStruct((N, 1), jnp.float32),
        ],
    )(aggp2, degp, h, xd, *weights)


def _prep_weights(params):
    p = params
    w1l = p["sage1_l"]["W"].T
    w1r = p["sage1_r"]["W"].T
    b1 = (p["sage1_l"]["b"] + p["sage1_r"]["b"])[None, :]
    w2l = p["sage2_l"]["W"].T
    w2r = p["sage2_r"]["W"].T
    b2 = (p["sage2_l"]["b"] + p["sage2_r"]["b"])[None, :]

    l1f, l1b = p["lstm"][0]["fwd"], p["lstm"][0]["bwd"]
    l2f, l2b = p["lstm"][1]["fwd"], p["lstm"][1]["bwd"]
    w1cat = jnp.concatenate([l1f["Wih"].T, l1b["Wih"].T], axis=1)
    b1cat = jnp.concatenate([l1f["bih"] + l1f["bhh"],
                             l1b["bih"] + l1b["bhh"]])[None, :]
    whh1f = l1f["Whh"].T
    whh1b = l1b["Whh"].T
    wih2f = l2f["Wih"].T
    b2f = (l2f["bih"] + l2f["bhh"])[None, :]
    whh2f = l2f["Whh"].T
    wih2b = l2b["Wih"].T
    b2b = (l2b["bih"] + l2b["bhh"])[None, :]

    sg = 1.0 / jnp.sqrt(jnp.float32(GNN_H))
    sr = 1.0 / jnp.sqrt(jnp.float32(RNN_DIM))
    wqg = p["q_gnn"]["W"].T * sg
    bqg = (p["q_gnn"]["b"] * sg)[None, :]
    wkr = p["k_rnn"]["W"].T
    bkr = p["k_rnn"]["b"][None, :]
    wvr = p["v_rnn"]["W"].T
    bvr = p["v_rnn"]["b"][None, :]
    wprg = p["proj_rnn_to_gnn"]["W"].T
    wqr = p["q_rnn"]["W"].T * sr
    bqr = (p["q_rnn"]["b"] * sr)[None, :]
    wkg = p["k_gnn"]["W"].T
    bkg = p["k_gnn"]["b"][None, :]
    wvg = p["v_gnn"]["W"].T
    bvg = p["v_gnn"]["b"][None, :]
    wpgr = p["proj_gnn_to_rnn"]["W"].T

    s1 = p["bn1"]["gamma"] / jnp.sqrt(p["bn1"]["var"] + 1e-5)
    w1m = p["mlp1"]["W"].T * s1[None, :]
    b1m = ((p["mlp1"]["b"] - p["bn1"]["mean"]) * s1 + p["bn1"]["beta"])[None, :]
    s2 = p["bn2"]["gamma"] / jnp.sqrt(p["bn2"]["var"] + 1e-5)
    w2m = p["mlp2"]["W"].T * s2[None, :]
    b2m = ((p["mlp2"]["b"] - p["bn2"]["mean"]) * s2 + p["bn2"]["beta"])[None, :]
    w3m = jnp.zeros((64, 128), jnp.float32).at[:, 0].set(p["mlp3"]["W"][0])
    b3m = jnp.zeros((1, 128), jnp.float32).at[0, 0].set(p["mlp3"]["b"][0])

    sage1 = (w1l, w1r, b1)
    main = (w2l, w2r, b2,
            w1cat, b1cat, whh1f, whh1b,
            wih2f, b2f, whh2f, wih2b, b2b,
            wqg, bqg, wkr, bkr, wvr, bvr, wprg,
            wqr, bqr, wkg, bkg, wvg, bvg, wpgr,
            w1m, b1m, w2m, b2m, w3m, b3m)
    return sage1, main


def _stack_halves(x):
    return jnp.stack([x[:, :HALF], x[:, HALF:]])


def kernel(x_static, x_dynamic, edge_index, params):
    src3d = edge_index[0].reshape(NS, NCHUNK, CH)
    dst3d = edge_index[1].reshape(NS, NCHUNK, CH)
    sage1_w, main_w = _prep_weights(params)

    degp = _sc_degree(dst3d)
    degp = degp[:, :N].T
    aggp1 = _sc_segsum(_stack_halves(x_static), src3d, dst3d)
    h = _tc_sage1(aggp1, degp, x_static, *sage1_w)
    aggp2 = _sc_segsum(_stack_halves(h), src3d, dst3d)
    xd = x_dynamic.reshape(N, T * DYN_DIM)
    logits, p0, p1 = _tc_main(aggp2, degp, h, xd, main_w)
    attn = jnp.concatenate([p0, p1], axis=1)
    return logits[:, 0], attn


# speedup vs baseline: 5.7491x; 1.0857x over previous
import functools

import jax
import jax.numpy as jnp
from jax import lax
from jax.experimental import pallas as pl
from jax.experimental.pallas import tpu as pltpu, tpu_sc as plsc

N = 10000
E = 320000
STATIC_DIM = 128
DYN_DIM = 16
T = 24
GNN_H = 128
RNN_H = 64
RNN_DIM = 2 * RNN_H
FUSED = GNN_H + RNN_DIM
EPS = 1e-12

NC = 2
NS = 16
NW = NC * NS
EPS_SUB = E // NS
CH = 80
NCHUNK = EPS_SUB // CH
NPAD = 10240
ROWS_PER_TILE = NPAD // NS
DEG_PAD = 10240
HALF = 64

_SC_MESH = dict(core_axis_name="c", subcore_axis_name="s",
                num_cores=NC, num_subcores=NS)


def _zero_vmem_2d(ref, nrows, ncols):
    z = jnp.zeros((16,), jnp.float32)

    def body(i, _):
        r = i // (ncols // 16)
        k = i % (ncols // 16)
        ref[r, pl.ds(k * 16, 16)] = z
        return 0

    lax.fori_loop(0, nrows * (ncols // 16), body, 0)


def _zero_vmem_1d(ref, n):
    z = jnp.zeros((16,), jnp.float32)

    def body(i, _):
        ref[pl.ds(i * 16, 16)] = z
        return 0

    lax.fori_loop(0, n // 16, body, 0)


def _segsum_body(xs_hbm, src_hbm, dst_hbm, agg_hbm,
                 idx_src, idx_dst, rows0, rows1, zbuf, agg_sh, sem0, sem1):
    c = lax.axis_index("c")
    s = lax.axis_index("s")

    _zero_vmem_2d(zbuf, CH, HALF)
    for r in range(ROWS_PER_TILE // CH):
        pltpu.sync_copy(zbuf, agg_sh.at[pl.ds(s * ROWS_PER_TILE + r * CH, CH)])
    plsc.subcore_barrier()

    pltpu.sync_copy(src_hbm.at[s], idx_src)
    pltpu.sync_copy(dst_hbm.at[s], idx_dst)

    xh = xs_hbm.at[c]
    pltpu.async_copy(xh.at[idx_src.at[0]], rows0, sem0)

    def pair(p, _):
        j0 = 2 * p
        j1 = j0 + 1
        j2 = j0 + 2
        pltpu.async_copy(xh.at[idx_src.at[j1]], rows1, sem1)
        pltpu.make_async_copy(xh.at[idx_src.at[j0]], rows0, sem0).wait()
        pltpu.sync_copy(rows0, agg_sh.at[idx_dst.at[j0]], add=True)

        @pl.when(j2 < NCHUNK)
        def _():
            pltpu.async_copy(xh.at[idx_src.at[j2]], rows0, sem0)

        pltpu.make_async_copy(xh.at[idx_src.at[j1]], rows1, sem1).wait()
        pltpu.sync_copy(rows1, agg_sh.at[idx_dst.at[j1]], add=True)
        return 0

    lax.fori_loop(0, NCHUNK // 2, pair, 0)

    plsc.subcore_barrier()
    pltpu.sync_copy(agg_sh.at[pl.ds(s * ROWS_PER_TILE, ROWS_PER_TILE)],
                    agg_hbm.at[c, pl.ds(s * ROWS_PER_TILE, ROWS_PER_TILE)])


def _sc_segsum(xs, src3d, dst3d):
    k = pl.kernel(
        _segsum_body,
        out_type=jax.ShapeDtypeStruct((NC, NPAD, HALF), jnp.float32),
        mesh=plsc.VectorSubcoreMesh(**_SC_MESH),
        compiler_params=pltpu.CompilerParams(use_tc_tiling_on_sc=False),
        scratch_types=[
            pltpu.VMEM((NCHUNK, CH), jnp.int32),
            pltpu.VMEM((NCHUNK, CH), jnp.int32),
            pltpu.VMEM((CH, HALF), jnp.float32),
            pltpu.VMEM((CH, HALF), jnp.float32),
            pltpu.VMEM((CH, HALF), jnp.float32),
            pltpu.VMEM_SHARED((NPAD, HALF), jnp.float32),
            pltpu.SemaphoreType.DMA,
            pltpu.SemaphoreType.DMA,
        ],
    )
    return k(xs, src3d, dst3d)


def _deg_body(dst_hbm, deg_hbm, idx_dst, ones_v, zb, deg_sh):
    c = lax.axis_index("c")
    s = lax.axis_index("s")

    _zero_vmem_1d(zb, DEG_PAD // NS)
    pltpu.sync_copy(zb, deg_sh.at[pl.ds(s * (DEG_PAD // NS), DEG_PAD // NS)])

    def ones_init(i, _):
        ones_v[pl.ds(i * 16, 16)] = jnp.ones((16,), jnp.float32)
        return 0

    lax.fori_loop(0, CH // 16, ones_init, 0)
    plsc.subcore_barrier()

    pltpu.sync_copy(dst_hbm.at[s, pl.ds(c * (NCHUNK // 2), NCHUNK // 2)],
                    idx_dst)

    def body(j, _):
        pltpu.sync_copy(ones_v, deg_sh.at[idx_dst.at[j]], add=True)
        return 0

    lax.fori_loop(0, NCHUNK // 2, body, 0)
    plsc.subcore_barrier()
    pltpu.sync_copy(deg_sh.at[pl.ds(s * (DEG_PAD // NS), DEG_PAD // NS)],
                    deg_hbm.at[c, pl.ds(s * (DEG_PAD // NS), DEG_PAD // NS)])


def _sc_degree(dst3d):
    k = pl.kernel(
        _deg_body,
        out_type=jax.ShapeDtypeStruct((NC, DEG_PAD), jnp.float32),
        mesh=plsc.VectorSubcoreMesh(**_SC_MESH),
        compiler_params=pltpu.CompilerParams(use_tc_tiling_on_sc=False),
        scratch_types=[
            pltpu.VMEM((NCHUNK // 2, CH), jnp.int32),
            pltpu.VMEM((CH,), jnp.float32),
            pltpu.VMEM((DEG_PAD // NS,), jnp.float32),
            pltpu.VMEM_SHARED((DEG_PAD,), jnp.float32),
        ],
    )
    return k(dst3d)


BN = 400


def _dot(a, b):
    return jnp.dot(a, b, preferred_element_type=jnp.float32)


def _sage1_tc_body(aggp_ref, degp_ref, x_ref,
                   wl_ref, wr_ref, b_ref, h_ref):
    agg = jnp.concatenate([aggp_ref[0], aggp_ref[1]], axis=1)
    deg = degp_ref[:, 0] + degp_ref[:, 1]
    inv = 1.0 / jnp.maximum(deg, 1.0)
    mean = agg * inv[:, None]
    out = _dot(mean, wl_ref[...]) + _dot(x_ref[...], wr_ref[...]) + b_ref[...]
    nrm = jnp.sqrt(jnp.sum(out * out, axis=1, keepdims=True))
    out = out / jnp.maximum(nrm, EPS)
    h_ref[...] = jnp.maximum(out, 0.0)


def _tc_sage1(aggp, degp, x, wl, wr, b):
    full = lambda a: pl.BlockSpec(a.shape, lambda i: (0,) * a.ndim)
    return pl.pallas_call(
        _sage1_tc_body,
        grid=(N // BN,),
        in_specs=[
            pl.BlockSpec((NC, BN, HALF), lambda i: (0, i, 0)),
            pl.BlockSpec((BN, NC), lambda i: (i, 0)),
            pl.BlockSpec((BN, STATIC_DIM), lambda i: (i, 0)),
            full(wl), full(wr), full(b),
        ],
        out_specs=pl.BlockSpec((BN, GNN_H), lambda i: (i, 0)),
        out_shape=jax.ShapeDtypeStruct((N, GNN_H), jnp.float32),
    )(aggp, degp, x, wl, wr, b)


def _dot16(a, b16):
    return jnp.dot(a.astype(jnp.bfloat16), b16,
                   preferred_element_type=jnp.float32)


def _lstm_step(g, h, c, whh_t):
    g = g + _dot16(h, whh_t)
    i = jax.nn.sigmoid(g[:, 0:RNN_H])
    f = jax.nn.sigmoid(g[:, RNN_H:2 * RNN_H])
    gg = jnp.tanh(g[:, 2 * RNN_H:3 * RNN_H])
    o = jax.nn.sigmoid(g[:, 3 * RNN_H:4 * RNN_H])
    c = f * c + i * gg
    h = o * jnp.tanh(c)
    return h, c


def _bilstm_tc_body(xd_ref,
                    w1cat_ref, b1cat_ref, whh1f_ref, whh1b_ref,
                    wih2f_ref, b2f_ref, whh2f_ref, wih2b_ref, b2b_ref,
                    hrnn_ref):
    xblk = xd_ref[...].astype(jnp.bfloat16)
    w1cat = w1cat_ref[...]
    b1cat = b1cat_ref[...]
    g1 = [
        jnp.dot(xblk[:, t * DYN_DIM:(t + 1) * DYN_DIM], w1cat,
                preferred_element_type=jnp.float32) + b1cat
        for t in range(T)
    ]
    zero = jnp.zeros((xblk.shape[0], RNN_H), jnp.float32)

    whh1b = whh1b_ref[...]
    hb, cb = zero, zero
    hb_seq = [None] * T
    for t in range(T - 1, -1, -1):
        hb, cb = _lstm_step(g1[t][:, 4 * RNN_H:], hb, cb, whh1b)
        hb_seq[t] = hb

    whh1f = whh1f_ref[...]
    wih2f = wih2f_ref[...]
    b2f = b2f_ref[...]
    whh2f = whh2f_ref[...]
    hf, cf = zero, zero
    h2, c2 = zero, zero
    for t in range(T):
        hf, cf = _lstm_step(g1[t][:, :4 * RNN_H], hf, cf, whh1f)
        x2t = jnp.concatenate([hf, hb_seq[t]], axis=1)
        g2 = _dot16(x2t, wih2f) + b2f
        h2, c2 = _lstm_step(g2, h2, c2, whh2f)
    x2_last = jnp.concatenate([hf, hb_seq[T - 1]], axis=1)
    g2b = _dot16(x2_last, wih2b_ref[...]) + b2b_ref[...]
    ib = jax.nn.sigmoid(g2b[:, 0:RNN_H])
    ggb = jnp.tanh(g2b[:, 2 * RNN_H:3 * RNN_H])
    ob = jax.nn.sigmoid(g2b[:, 3 * RNN_H:4 * RNN_H])
    c2b = ib * ggb
    h2b = ob * jnp.tanh(c2b)
    hrnn_ref[...] = jnp.concatenate([h2, h2b], axis=1)


def _tc_bilstm(xd, weights):
    full = lambda a: pl.BlockSpec(a.shape, lambda i: (0,) * a.ndim)
    return pl.pallas_call(
        _bilstm_tc_body,
        grid=(N // BN,),
        in_specs=[pl.BlockSpec((BN, T * DYN_DIM), lambda i: (i, 0))]
        + [full(w) for w in weights],
        out_specs=pl.BlockSpec((BN, RNN_DIM), lambda i: (i, 0)),
        out_shape=jax.ShapeDtypeStruct((N, RNN_DIM), jnp.float32),
    )(xd, *weights)


def _main_tc_body(aggp_ref, degp_ref, h_ref, hrnn_ref,
                  w2l_ref, w2r_ref, b2_ref,
                  wqg_ref, bqg_ref, wkr_ref, bkr_ref, wvr_ref, bvr_ref,
                  wprg_ref,
                  wqr_ref, bqr_ref, wkg_ref, bkg_ref, wvg_ref, bvg_ref,
                  wpgr_ref,
                  w1m_ref, b1m_ref, w2m_ref, b2m_ref, w3m_ref, b3m_ref,
                  logits_ref, p0_ref, p1_ref):
    h = h_ref[...]
    h_rnn = hrnn_ref[...]
    agg = jnp.concatenate([aggp_ref[0], aggp_ref[1]], axis=1)
    deg = degp_ref[:, 0] + degp_ref[:, 1]
    inv = 1.0 / jnp.maximum(deg, 1.0)
    mean = agg * inv[:, None]
    g = _dot(mean, w2l_ref[...]) + _dot(h, w2r_ref[...]) + b2_ref[...]
    nrm = jnp.sqrt(jnp.sum(g * g, axis=1, keepdims=True))
    h_gnn = g / jnp.maximum(nrm, EPS)

    q_gnn = _dot(h_gnn, wqg_ref[...]) + bqg_ref[...]
    k_rnn = _dot(h_rnn, wkr_ref[...]) + bkr_ref[...]
    v_rnn = _dot(h_rnn, wvr_ref[...]) + bvr_ref[...]
    s_g = jnp.sum(q_gnn * k_rnn, axis=1, keepdims=True)
    a_g = jax.nn.sigmoid(s_g)
    h_gnn_att = h_gnn + a_g * _dot(v_rnn, wprg_ref[...])

    q_rnn = _dot(h_rnn, wqr_ref[...]) + bqr_ref[...]
    k_gnn = _dot(h_gnn, wkg_ref[...]) + bkg_ref[...]
    v_gnn = _dot(h_gnn, wvg_ref[...]) + bvg_ref[...]
    s_r = jnp.sum(q_rnn * k_gnn, axis=1, keepdims=True)
    a_r = jax.nn.sigmoid(s_r)
    h_rnn_att = h_rnn + a_r * _dot(v_gnn, wpgr_ref[...])

    p0 = jax.nn.sigmoid(a_g - a_r)
    p0_ref[...] = p0
    p1_ref[...] = 1.0 - p0

    z = jnp.concatenate([h_gnn_att, h_rnn_att], axis=1)
    z = jnp.maximum(_dot(z, w1m_ref[...]) + b1m_ref[...], 0.0)
    z = jnp.maximum(_dot(z, w2m_ref[...]) + b2m_ref[...], 0.0)
    logits = _dot(z, w3m_ref[...]) + b3m_ref[...]
    logits_ref[...] = logits[:, 0:1]


def _tc_main(aggp2, degp, h, hrnn, weights):
    full = lambda a: pl.BlockSpec(a.shape, lambda i: (0,) * a.ndim)
    in_specs = [
        pl.BlockSpec((NC, BN, HALF), lambda i: (0, i, 0)),
        pl.BlockSpec((BN, NC), lambda i: (i, 0)),
        pl.BlockSpec((BN, GNN_H), lambda i: (i, 0)),
        pl.BlockSpec((BN, RNN_DIM), lambda i: (i, 0)),
    ] + [full(w) for w in weights]
    return pl.pallas_call(
        _main_tc_body,
        grid=(N // BN,),
        in_specs=in_specs,
        out_specs=[
            pl.BlockSpec((BN, 1), lambda i: (i, 0)),
            pl.BlockSpec((BN, 1), lambda i: (i, 0)),
            pl.BlockSpec((BN, 1), lambda i: (i, 0)),
        ],
        out_shape=[
            jax.ShapeDtypeStruct((N, 1), jnp.float32),
            jax.ShapeDtypeStruct((N, 1), jnp.float32),
            jax.ShapeDtypeStruct((N, 1), jnp.float32),
        ],
    )(aggp2, degp, h, hrnn, *weights)


def _prep_weights(params):
    p = params
    w1l = p["sage1_l"]["W"].T
    w1r = p["sage1_r"]["W"].T
    b1 = (p["sage1_l"]["b"] + p["sage1_r"]["b"])[None, :]
    w2l = p["sage2_l"]["W"].T
    w2r = p["sage2_r"]["W"].T
    b2 = (p["sage2_l"]["b"] + p["sage2_r"]["b"])[None, :]

    l1f, l1b = p["lstm"][0]["fwd"], p["lstm"][0]["bwd"]
    l2f, l2b = p["lstm"][1]["fwd"], p["lstm"][1]["bwd"]
    w1cat = jnp.concatenate([l1f["Wih"].T, l1b["Wih"].T], axis=1)
    b1cat = jnp.concatenate([l1f["bih"] + l1f["bhh"],
                             l1b["bih"] + l1b["bhh"]])[None, :]
    whh1f = l1f["Whh"].T
    whh1b = l1b["Whh"].T
    wih2f = l2f["Wih"].T
    b2f = (l2f["bih"] + l2f["bhh"])[None, :]
    whh2f = l2f["Whh"].T
    wih2b = l2b["Wih"].T
    b2b = (l2b["bih"] + l2b["bhh"])[None, :]

    sg = 1.0 / jnp.sqrt(jnp.float32(GNN_H))
    sr = 1.0 / jnp.sqrt(jnp.float32(RNN_DIM))
    wqg = p["q_gnn"]["W"].T * sg
    bqg = (p["q_gnn"]["b"] * sg)[None, :]
    wkr = p["k_rnn"]["W"].T
    bkr = p["k_rnn"]["b"][None, :]
    wvr = p["v_rnn"]["W"].T
    bvr = p["v_rnn"]["b"][None, :]
    wprg = p["proj_rnn_to_gnn"]["W"].T
    wqr = p["q_rnn"]["W"].T * sr
    bqr = (p["q_rnn"]["b"] * sr)[None, :]
    wkg = p["k_gnn"]["W"].T
    bkg = p["k_gnn"]["b"][None, :]
    wvg = p["v_gnn"]["W"].T
    bvg = p["v_gnn"]["b"][None, :]
    wpgr = p["proj_gnn_to_rnn"]["W"].T

    s1 = p["bn1"]["gamma"] / jnp.sqrt(p["bn1"]["var"] + 1e-5)
    w1m = p["mlp1"]["W"].T * s1[None, :]
    b1m = ((p["mlp1"]["b"] - p["bn1"]["mean"]) * s1 + p["bn1"]["beta"])[None, :]
    s2 = p["bn2"]["gamma"] / jnp.sqrt(p["bn2"]["var"] + 1e-5)
    w2m = p["mlp2"]["W"].T * s2[None, :]
    b2m = ((p["mlp2"]["b"] - p["bn2"]["mean"]) * s2 + p["bn2"]["beta"])[None, :]
    w3m = jnp.zeros((64, 128), jnp.float32).at[:, 0].set(p["mlp3"]["W"][0])
    b3m = jnp.zeros((1, 128), jnp.float32).at[0, 0].set(p["mlp3"]["b"][0])

    bf = lambda a: a.astype(jnp.bfloat16)
    sage1 = (w1l, w1r, b1)
    lstm = (bf(w1cat), b1cat, bf(whh1f), bf(whh1b),
            bf(wih2f), b2f, bf(whh2f), bf(wih2b), b2b)
    main = (w2l, w2r, b2,
            wqg, bqg, wkr, bkr, wvr, bvr, wprg,
            wqr, bqr, wkg, bkg, wvg, bvg, wpgr,
            w1m, b1m, w2m, b2m, w3m, b3m)
    return sage1, lstm, main


def _stack_halves(x):
    return jnp.stack([x[:, :HALF], x[:, HALF:]])


def kernel(x_static, x_dynamic, edge_index, params):
    src3d = edge_index[0].reshape(NS, NCHUNK, CH)
    dst3d = edge_index[1].reshape(NS, NCHUNK, CH)
    sage1_w, lstm_w, main_w = _prep_weights(params)

    degp = _sc_degree(dst3d)
    degp = degp[:, :N].T
    aggp1 = _sc_segsum(_stack_halves(x_static), src3d, dst3d)
    xd = x_dynamic.reshape(N, T * DYN_DIM)
    hrnn = _tc_bilstm(xd, lstm_w)
    h = _tc_sage1(aggp1, degp, x_static, *sage1_w)
    aggp2 = _sc_segsum(_stack_halves(h), src3d, dst3d)
    logits, p0, p1 = _tc_main(aggp2, degp, h, hrnn, main_w)
    attn = jnp.concatenate([p0, p1], axis=1)
    return logits[:, 0], attn


# speedup vs baseline: 6.0368x; 1.0500x over previous
import functools

import jax
import jax.numpy as jnp
from jax import lax
from jax.experimental import pallas as pl
from jax.experimental.pallas import tpu as pltpu, tpu_sc as plsc

N = 10000
E = 320000
STATIC_DIM = 128
DYN_DIM = 16
T = 24
GNN_H = 128
RNN_H = 64
RNN_DIM = 2 * RNN_H
FUSED = GNN_H + RNN_DIM
EPS = 1e-12

NC = 2
NS = 16
NW = NC * NS
EPS_SUB = E // NS
CH = 80
NCHUNK = EPS_SUB // CH
NPAD = 10240
ROWS_PER_TILE = NPAD // NS
DEG_PAD = 10240
HALF = 64

_SC_MESH = dict(core_axis_name="c", subcore_axis_name="s",
                num_cores=NC, num_subcores=NS)


def _zero_vmem_2d(ref, nrows, ncols):
    z = jnp.zeros((16,), jnp.float32)

    def body(i, _):
        r = i // (ncols // 16)
        k = i % (ncols // 16)
        ref[r, pl.ds(k * 16, 16)] = z
        return 0

    lax.fori_loop(0, nrows * (ncols // 16), body, 0)


def _zero_vmem_1d(ref, n):
    z = jnp.zeros((16,), jnp.float32)

    def body(i, _):
        ref[pl.ds(i * 16, 16)] = z
        return 0

    lax.fori_loop(0, n // 16, body, 0)


def _segsum_body(xs_hbm, src_hbm, dst_hbm, agg_hbm,
                 idx_src, idx_dst, rows0, rows1, zbuf, agg_sh, sem0, sem1):
    c = lax.axis_index("c")
    s = lax.axis_index("s")

    _zero_vmem_2d(zbuf, CH, HALF)
    for r in range(ROWS_PER_TILE // CH):
        pltpu.sync_copy(zbuf, agg_sh.at[pl.ds(s * ROWS_PER_TILE + r * CH, CH)])
    plsc.subcore_barrier()

    pltpu.sync_copy(src_hbm.at[s], idx_src)
    pltpu.sync_copy(dst_hbm.at[s], idx_dst)

    xh = xs_hbm.at[c]
    pltpu.async_copy(xh.at[idx_src.at[0]], rows0, sem0)

    def pair(p, _):
        j0 = 2 * p
        j1 = j0 + 1
        j2 = j0 + 2
        pltpu.async_copy(xh.at[idx_src.at[j1]], rows1, sem1)
        pltpu.make_async_copy(xh.at[idx_src.at[j0]], rows0, sem0).wait()
        pltpu.sync_copy(rows0, agg_sh.at[idx_dst.at[j0]], add=True)

        @pl.when(j2 < NCHUNK)
        def _():
            pltpu.async_copy(xh.at[idx_src.at[j2]], rows0, sem0)

        pltpu.make_async_copy(xh.at[idx_src.at[j1]], rows1, sem1).wait()
        pltpu.sync_copy(rows1, agg_sh.at[idx_dst.at[j1]], add=True)
        return 0

    lax.fori_loop(0, NCHUNK // 2, pair, 0)

    plsc.subcore_barrier()
    pltpu.sync_copy(agg_sh.at[pl.ds(s * ROWS_PER_TILE, ROWS_PER_TILE)],
                    agg_hbm.at[c, pl.ds(s * ROWS_PER_TILE, ROWS_PER_TILE)])


def _sc_segsum(xs, src3d, dst3d):
    k = pl.kernel(
        _segsum_body,
        out_type=jax.ShapeDtypeStruct((NC, NPAD, HALF), jnp.float32),
        mesh=plsc.VectorSubcoreMesh(**_SC_MESH),
        compiler_params=pltpu.CompilerParams(use_tc_tiling_on_sc=False),
        scratch_types=[
            pltpu.VMEM((NCHUNK, CH), jnp.int32),
            pltpu.VMEM((NCHUNK, CH), jnp.int32),
            pltpu.VMEM((CH, HALF), jnp.float32),
            pltpu.VMEM((CH, HALF), jnp.float32),
            pltpu.VMEM((CH, HALF), jnp.float32),
            pltpu.VMEM_SHARED((NPAD, HALF), jnp.float32),
            pltpu.SemaphoreType.DMA,
            pltpu.SemaphoreType.DMA,
        ],
    )
    return k(xs, src3d, dst3d)


def _deg_body(dst_hbm, deg_hbm, idx_dst, ones_v, zb, deg_sh):
    c = lax.axis_index("c")
    s = lax.axis_index("s")

    _zero_vmem_1d(zb, DEG_PAD // NS)
    pltpu.sync_copy(zb, deg_sh.at[pl.ds(s * (DEG_PAD // NS), DEG_PAD // NS)])

    def ones_init(i, _):
        ones_v[pl.ds(i * 16, 16)] = jnp.ones((16,), jnp.float32)
        return 0

    lax.fori_loop(0, CH // 16, ones_init, 0)
    plsc.subcore_barrier()

    pltpu.sync_copy(dst_hbm.at[s, pl.ds(c * (NCHUNK // 2), NCHUNK // 2)],
                    idx_dst)

    def body(j, _):
        pltpu.sync_copy(ones_v, deg_sh.at[idx_dst.at[j]], add=True)
        return 0

    lax.fori_loop(0, NCHUNK // 2, body, 0)
    plsc.subcore_barrier()
    pltpu.sync_copy(deg_sh.at[pl.ds(s * (DEG_PAD // NS), DEG_PAD // NS)],
                    deg_hbm.at[c, pl.ds(s * (DEG_PAD // NS), DEG_PAD // NS)])


def _sc_degree(dst3d):
    k = pl.kernel(
        _deg_body,
        out_type=jax.ShapeDtypeStruct((NC, DEG_PAD), jnp.float32),
        mesh=plsc.VectorSubcoreMesh(**_SC_MESH),
        compiler_params=pltpu.CompilerParams(use_tc_tiling_on_sc=False),
        scratch_types=[
            pltpu.VMEM((NCHUNK // 2, CH), jnp.int32),
            pltpu.VMEM((CH,), jnp.float32),
            pltpu.VMEM((DEG_PAD // NS,), jnp.float32),
            pltpu.VMEM_SHARED((DEG_PAD,), jnp.float32),
        ],
    )
    return k(dst3d)


BN = 400


def _dot(a, b):
    return jnp.dot(a, b, preferred_element_type=jnp.float32)


def _sage1_tc_body(aggp_ref, degp_ref, x_ref,
                   wl_ref, wr_ref, b_ref, h_ref):
    agg = jnp.concatenate([aggp_ref[0], aggp_ref[1]], axis=1)
    deg = degp_ref[:, 0] + degp_ref[:, 1]
    inv = 1.0 / jnp.maximum(deg, 1.0)
    mean = agg * inv[:, None]
    out = _dot(mean, wl_ref[...]) + _dot(x_ref[...], wr_ref[...]) + b_ref[...]
    nrm = jnp.sqrt(jnp.sum(out * out, axis=1, keepdims=True))
    out = out / jnp.maximum(nrm, EPS)
    h_ref[...] = jnp.maximum(out, 0.0)


def _tc_sage1(aggp, degp, x, wl, wr, b):
    full = lambda a: pl.BlockSpec(a.shape, lambda i: (0,) * a.ndim)
    return pl.pallas_call(
        _sage1_tc_body,
        grid=(N // BN,),
        in_specs=[
            pl.BlockSpec((NC, BN, HALF), lambda i: (0, i, 0)),
            pl.BlockSpec((BN, NC), lambda i: (i, 0)),
            pl.BlockSpec((BN, STATIC_DIM), lambda i: (i, 0)),
            full(wl), full(wr), full(b),
        ],
        out_specs=pl.BlockSpec((BN, GNN_H), lambda i: (i, 0)),
        out_shape=jax.ShapeDtypeStruct((N, GNN_H), jnp.float32),
    )(aggp, degp, x, wl, wr, b)


def _dot16(a, b16):
    return jnp.dot(a.astype(jnp.bfloat16), b16,
                   preferred_element_type=jnp.float32)


def _gates(g, c):
    i = jax.nn.sigmoid(g[:, 0:RNN_H])
    f = jax.nn.sigmoid(g[:, RNN_H:2 * RNN_H])
    gg = jnp.tanh(g[:, 2 * RNN_H:3 * RNN_H])
    o = jax.nn.sigmoid(g[:, 3 * RNN_H:4 * RNN_H])
    c = f * c + i * gg
    h = o * jnp.tanh(c)
    return h, c


BN_L = 1000


def _bilstm_tc_body(xd_ref,
                    w1f_ref, b1f_ref, w1b_ref, b1b_ref,
                    w2f_ref, b2f_ref, w2b_ref, b2b_ref,
                    hrnn_ref):
    xblk = xd_ref[...].astype(jnp.bfloat16)
    xts = [xblk[:, t * DYN_DIM:(t + 1) * DYN_DIM] for t in range(T)]
    zero = jnp.zeros((xblk.shape[0], RNN_H), jnp.float32)
    bf = lambda a: a.astype(jnp.bfloat16)

    w1f = w1f_ref[...]
    w1b = w1b_ref[...]
    b1f = b1f_ref[...]
    b1b = b1b_ref[...]
    hf, cf = zero, zero
    hb, cb = zero, zero
    hf_seq = [None] * T
    hb_seq = [None] * T
    for i in range(T):
        tb = T - 1 - i
        af = jnp.concatenate([xts[i], bf(hf)], axis=1)
        ab = jnp.concatenate([xts[tb], bf(hb)], axis=1)
        gf = jnp.dot(af, w1f, preferred_element_type=jnp.float32) + b1f
        gb = jnp.dot(ab, w1b, preferred_element_type=jnp.float32) + b1b
        hf, cf = _gates(gf, cf)
        hb, cb = _gates(gb, cb)
        hf_seq[i] = hf
        hb_seq[tb] = hb

    w2f = w2f_ref[...]
    b2f = b2f_ref[...]
    h2, c2 = zero, zero
    for t in range(T):
        a2 = jnp.concatenate([bf(hf_seq[t]), bf(hb_seq[t]), bf(h2)], axis=1)
        g2 = jnp.dot(a2, w2f, preferred_element_type=jnp.float32) + b2f
        h2, c2 = _gates(g2, c2)
    a2b = jnp.concatenate([bf(hf_seq[T - 1]), bf(hb_seq[T - 1])], axis=1)
    g2b = jnp.dot(a2b, w2b_ref[...],
                  preferred_element_type=jnp.float32) + b2b_ref[...]
    ib = jax.nn.sigmoid(g2b[:, 0:RNN_H])
    ggb = jnp.tanh(g2b[:, 2 * RNN_H:3 * RNN_H])
    ob = jax.nn.sigmoid(g2b[:, 3 * RNN_H:4 * RNN_H])
    c2b = ib * ggb
    h2b = ob * jnp.tanh(c2b)
    hrnn_ref[...] = jnp.concatenate([h2, h2b], axis=1)


def _tc_bilstm(xd, weights):
    full = lambda a: pl.BlockSpec(a.shape, lambda i: (0,) * a.ndim)
    return pl.pallas_call(
        _bilstm_tc_body,
        grid=(N // BN_L,),
        in_specs=[pl.BlockSpec((BN_L, T * DYN_DIM), lambda i: (i, 0))]
        + [full(w) for w in weights],
        out_specs=pl.BlockSpec((BN_L, RNN_DIM), lambda i: (i, 0)),
        out_shape=jax.ShapeDtypeStruct((N, RNN_DIM), jnp.float32),
    )(xd, *weights)


def _main_tc_body(aggp_ref, degp_ref, h_ref, hrnn_ref,
                  w2l_ref, w2r_ref, b2_ref,
                  wqg_ref, bqg_ref, wkr_ref, bkr_ref, wvr_ref, bvr_ref,
                  wprg_ref,
                  wqr_ref, bqr_ref, wkg_ref, bkg_ref, wvg_ref, bvg_ref,
                  wpgr_ref,
                  w1m_ref, b1m_ref, w2m_ref, b2m_ref, w3m_ref, b3m_ref,
                  logits_ref, p0_ref, p1_ref):
    h = h_ref[...]
    h_rnn = hrnn_ref[...]
    agg = jnp.concatenate([aggp_ref[0], aggp_ref[1]], axis=1)
    deg = degp_ref[:, 0] + degp_ref[:, 1]
    inv = 1.0 / jnp.maximum(deg, 1.0)
    mean = agg * inv[:, None]
    g = _dot(mean, w2l_ref[...]) + _dot(h, w2r_ref[...]) + b2_ref[...]
    nrm = jnp.sqrt(jnp.sum(g * g, axis=1, keepdims=True))
    h_gnn = g / jnp.maximum(nrm, EPS)

    q_gnn = _dot(h_gnn, wqg_ref[...]) + bqg_ref[...]
    k_rnn = _dot(h_rnn, wkr_ref[...]) + bkr_ref[...]
    v_rnn = _dot(h_rnn, wvr_ref[...]) + bvr_ref[...]
    s_g = jnp.sum(q_gnn * k_rnn, axis=1, keepdims=True)
    a_g = jax.nn.sigmoid(s_g)
    h_gnn_att = h_gnn + a_g * _dot(v_rnn, wprg_ref[...])

    q_rnn = _dot(h_rnn, wqr_ref[...]) + bqr_ref[...]
    k_gnn = _dot(h_gnn, wkg_ref[...]) + bkg_ref[...]
    v_gnn = _dot(h_gnn, wvg_ref[...]) + bvg_ref[...]
    s_r = jnp.sum(q_rnn * k_gnn, axis=1, keepdims=True)
    a_r = jax.nn.sigmoid(s_r)
    h_rnn_att = h_rnn + a_r * _dot(v_gnn, wpgr_ref[...])

    p0 = jax.nn.sigmoid(a_g - a_r)
    p0_ref[...] = p0
    p1_ref[...] = 1.0 - p0

    z = jnp.concatenate([h_gnn_att, h_rnn_att], axis=1)
    z = jnp.maximum(_dot(z, w1m_ref[...]) + b1m_ref[...], 0.0)
    z = jnp.maximum(_dot(z, w2m_ref[...]) + b2m_ref[...], 0.0)
    logits = _dot(z, w3m_ref[...]) + b3m_ref[...]
    logits_ref[...] = logits[:, 0:1]


def _tc_main(aggp2, degp, h, hrnn, weights):
    full = lambda a: pl.BlockSpec(a.shape, lambda i: (0,) * a.ndim)
    in_specs = [
        pl.BlockSpec((NC, BN, HALF), lambda i: (0, i, 0)),
        pl.BlockSpec((BN, NC), lambda i: (i, 0)),
        pl.BlockSpec((BN, GNN_H), lambda i: (i, 0)),
        pl.BlockSpec((BN, RNN_DIM), lambda i: (i, 0)),
    ] + [full(w) for w in weights]
    return pl.pallas_call(
        _main_tc_body,
        grid=(N // BN,),
        in_specs=in_specs,
        out_specs=[
            pl.BlockSpec((BN, 1), lambda i: (i, 0)),
            pl.BlockSpec((BN, 1), lambda i: (i, 0)),
            pl.BlockSpec((BN, 1), lambda i: (i, 0)),
        ],
        out_shape=[
            jax.ShapeDtypeStruct((N, 1), jnp.float32),
            jax.ShapeDtypeStruct((N, 1), jnp.float32),
            jax.ShapeDtypeStruct((N, 1), jnp.float32),
        ],
    )(aggp2, degp, h, hrnn, *weights)


def _prep_weights(params):
    p = params
    w1l = p["sage1_l"]["W"].T
    w1r = p["sage1_r"]["W"].T
    b1 = (p["sage1_l"]["b"] + p["sage1_r"]["b"])[None, :]
    w2l = p["sage2_l"]["W"].T
    w2r = p["sage2_r"]["W"].T
    b2 = (p["sage2_l"]["b"] + p["sage2_r"]["b"])[None, :]

    l1f, l1b = p["lstm"][0]["fwd"], p["lstm"][0]["bwd"]
    l2f, l2b = p["lstm"][1]["fwd"], p["lstm"][1]["bwd"]
    w1f = jnp.concatenate([l1f["Wih"].T, l1f["Whh"].T], axis=0)
    b1f = (l1f["bih"] + l1f["bhh"])[None, :]
    w1b = jnp.concatenate([l1b["Wih"].T, l1b["Whh"].T], axis=0)
    b1b = (l1b["bih"] + l1b["bhh"])[None, :]
    w2f = jnp.concatenate([l2f["Wih"].T, l2f["Whh"].T], axis=0)
    b2f = (l2f["bih"] + l2f["bhh"])[None, :]
    w2b = l2b["Wih"].T
    b2b = (l2b["bih"] + l2b["bhh"])[None, :]

    sg = 1.0 / jnp.sqrt(jnp.float32(GNN_H))
    sr = 1.0 / jnp.sqrt(jnp.float32(RNN_DIM))
    wqg = p["q_gnn"]["W"].T * sg
    bqg = (p["q_gnn"]["b"] * sg)[None, :]
    wkr = p["k_rnn"]["W"].T
    bkr = p["k_rnn"]["b"][None, :]
    wvr = p["v_rnn"]["W"].T
    bvr = p["v_rnn"]["b"][None, :]
    wprg = p["proj_rnn_to_gnn"]["W"].T
    wqr = p["q_rnn"]["W"].T * sr
    bqr = (p["q_rnn"]["b"] * sr)[None, :]
    wkg = p["k_gnn"]["W"].T
    bkg = p["k_gnn"]["b"][None, :]
    wvg = p["v_gnn"]["W"].T
    bvg = p["v_gnn"]["b"][None, :]
    wpgr = p["proj_gnn_to_rnn"]["W"].T

    s1 = p["bn1"]["gamma"] / jnp.sqrt(p["bn1"]["var"] + 1e-5)
    w1m = p["mlp1"]["W"].T * s1[None, :]
    b1m = ((p["mlp1"]["b"] - p["bn1"]["mean"]) * s1 + p["bn1"]["beta"])[None, :]
    s2 = p["bn2"]["gamma"] / jnp.sqrt(p["bn2"]["var"] + 1e-5)
    w2m = p["mlp2"]["W"].T * s2[None, :]
    b2m = ((p["mlp2"]["b"] - p["bn2"]["mean"]) * s2 + p["bn2"]["beta"])[None, :]
    w3m = jnp.zeros((64, 128), jnp.float32).at[:, 0].set(p["mlp3"]["W"][0])
    b3m = jnp.zeros((1, 128), jnp.float32).at[0, 0].set(p["mlp3"]["b"][0])

    bf = lambda a: a.astype(jnp.bfloat16)
    sage1 = (w1l, w1r, b1)
    lstm = (bf(w1f), b1f, bf(w1b), b1b,
            bf(w2f), b2f, bf(w2b), b2b)
    main = (w2l, w2r, b2,
            wqg, bqg, wkr, bkr, wvr, bvr, wprg,
            wqr, bqr, wkg, bkg, wvg, bvg, wpgr,
            w1m, b1m, w2m, b2m, w3m, b3m)
    return sage1, lstm, main


def _stack_halves(x):
    return jnp.stack([x[:, :HALF], x[:, HALF:]])


def kernel(x_static, x_dynamic, edge_index, params):
    src3d = edge_index[0].reshape(NS, NCHUNK, CH)
    dst3d = edge_index[1].reshape(NS, NCHUNK, CH)
    sage1_w, lstm_w, main_w = _prep_weights(params)

    degp = _sc_degree(dst3d)
    degp = degp[:, :N].T
    aggp1 = _sc_segsum(_stack_halves(x_static), src3d, dst3d)
    xd = x_dynamic.reshape(N, T * DYN_DIM)
    hrnn = _tc_bilstm(xd, lstm_w)
    h = _tc_sage1(aggp1, degp, x_static, *sage1_w)
    aggp2 = _sc_segsum(_stack_halves(h), src3d, dst3d)
    logits, p0, p1 = _tc_main(aggp2, degp, h, hrnn, main_w)
    attn = jnp.concatenate([p0, p1], axis=1)
    return logits[:, 0], attn
